# Initial kernel scaffold; baseline (speedup 1.0000x reference)
import jax
import jax.numpy as jnp
from jax.experimental import pallas as pl


def kernel(cont, params, cat):
    raise NotImplementedError("write your pallas kernel here")


# speedup vs baseline: 17.5532x; 17.5532x over previous
import functools

import jax
import jax.numpy as jnp
from jax import lax
from jax.experimental import pallas as pl
from jax.experimental.pallas import tpu as pltpu
from jax.experimental.pallas import tpu_sc as plsc

B = 4096
N_CONT = 13
N_CAT = 26
VOCAB = 1000
EMB = 16
D_MODEL = 128
N_EXPERTS = 8
TOP_K = 2
D_FF = 256
N_LAYERS = 2
N_CLASSES = 2
HIDDEN = max(D_FF, D_MODEL)

BT = 512
GRID = B // BT
NROWS = B * N_CAT


def _embed_gather_sc(table, idx):
    info = plsc.get_sparse_core_info()
    nw = info.num_cores * info.num_subcores
    b_per_w = NROWS // nw
    nc = info.num_cores
    mesh = plsc.VectorSubcoreMesh(core_axis_name="c", subcore_axis_name="s")

    @functools.partial(
        pl.kernel,
        mesh=mesh,
        out_type=jax.ShapeDtypeStruct((NROWS, EMB), jnp.float32),
        scratch_types=[
            pltpu.VMEM((b_per_w,), jnp.int32),
            pltpu.VMEM((b_per_w, EMB), jnp.float32),
            pltpu.SemaphoreType.DMA,
        ],
        compiler_params=pltpu.CompilerParams(use_tc_tiling_on_sc=False),
    )
    def gather(table_hbm, idx_hbm, out_hbm, idx_v, rows_v, sem):
        wid = lax.axis_index("s") * nc + lax.axis_index("c")
        base = wid * b_per_w
        pltpu.sync_copy(idx_hbm.at[pl.ds(base, b_per_w)], idx_v)
        pltpu.async_copy(table_hbm.at[idx_v], rows_v, sem).wait()
        pltpu.sync_copy(rows_v, out_hbm.at[pl.ds(base, b_per_w)])

    return gather(table, idx)


def _dot(a, b):
    return jax.lax.dot_general(
        a, b, (((a.ndim - 1,), (0,)), ((), ())),
        preferred_element_type=jnp.float32)


def _layer_norm(x, g, b):
    mu = jnp.mean(x, axis=-1, keepdims=True)
    xc = x - mu
    var = jnp.mean(xc * xc, axis=-1, keepdims=True)
    return xc * lax.rsqrt(var + 1e-5) * g + b


def _net_body(cont_ref, cate_ref, wpc_ref, wpe_ref, bp_ref, wv_ref, wo_ref,
              ln1g_ref, ln1b_ref, ln2g_ref, ln2b_ref, wr_ref, w1_ref, b1_ref,
              w2_ref, b2_ref, wc1_ref, bc1_ref, wc2_ref, bc2_ref,
              logits_ref, aux_ref, acc_ref):
    step = pl.program_id(0)

    @pl.when(step == 0)
    def _init():
        acc_ref[...] = jnp.zeros_like(acc_ref)

    z = (_dot(cont_ref[...], wpc_ref[...]) + _dot(cate_ref[...], wpe_ref[...])
         + bp_ref[...])
    for l in range(N_LAYERS):
        attn = _dot(_dot(z, wv_ref[l]), wo_ref[l])
        z = _layer_norm(z + attn, ln1g_ref[l], ln1b_ref[l])
        probs = jax.nn.softmax(_dot(z, wr_ref[l]), axis=-1)
        iota = lax.broadcasted_iota(jnp.int32, probs.shape, 1)
        m1 = jnp.max(probs, axis=-1, keepdims=True)
        i1 = jnp.min(jnp.where(probs == m1, iota, N_EXPERTS), axis=-1,
                     keepdims=True)
        oh1 = iota == i1
        p2 = jnp.where(oh1, -1.0, probs)
        m2 = jnp.max(p2, axis=-1, keepdims=True)
        i2 = jnp.min(jnp.where(p2 == m2, iota, N_EXPERTS), axis=-1,
                     keepdims=True)
        oh2 = iota == i2
        denom = m1 + m2 + 1e-9
        combine = (jnp.where(oh1, m1 / denom, 0.0)
                   + jnp.where(oh2, m2 / denom, 0.0))
        acc_ref[2 * l:2 * l + 1, :] += jnp.sum(probs, axis=0, keepdims=True)
        acc_ref[2 * l + 1:2 * l + 2, :] += jnp.sum(
            (combine > 0).astype(jnp.float32), axis=0, keepdims=True)
        moe = jnp.zeros_like(z)
        for e in range(N_EXPERTS):
            h = jax.nn.gelu(_dot(z, w1_ref[l, e]) + b1_ref[l, e])
            y = _dot(h, w2_ref[l, e]) + b2_ref[l, e]
            moe = moe + y * combine[:, e:e + 1]
        z = _layer_norm(z + moe, ln2g_ref[l], ln2b_ref[l])
    hc = jax.nn.gelu(_dot(z, wc1_ref[...]) + bc1_ref[...])
    logits_ref[...] = _dot(hc, wc2_ref[...]) + bc2_ref[...]

    @pl.when(step == GRID - 1)
    def _fin():
        scale = float(N_EXPERTS) / (B * B)
        a0 = jnp.sum(acc_ref[0:1, :] * acc_ref[1:2, :]) * scale
        a1 = jnp.sum(acc_ref[2:3, :] * acc_ref[3:4, :]) * scale
        aux_ref[...] = jnp.concatenate(
            [jnp.full((1, 1), a0, jnp.float32),
             jnp.full((1, 1), a1, jnp.float32)], axis=1)


def _net_tc(cont, cate, p):
    d = D_MODEL
    full = lambda *shape: pl.BlockSpec(shape, lambda i: (0,) * len(shape))
    in_specs = [
        pl.BlockSpec((BT, N_CONT), lambda i: (i, 0)),
        pl.BlockSpec((BT, N_CAT * EMB), lambda i: (i, 0)),
        full(N_CONT, d),
        full(N_CAT * EMB, d),
        full(1, d),
        full(N_LAYERS, d, d),
        full(N_LAYERS, d, d),
        full(N_LAYERS, 1, d),
        full(N_LAYERS, 1, d),
        full(N_LAYERS, 1, d),
        full(N_LAYERS, 1, d),
        full(N_LAYERS, d, N_EXPERTS),
        full(N_LAYERS, N_EXPERTS, d, D_FF),
        full(N_LAYERS, N_EXPERTS, 1, D_FF),
        full(N_LAYERS, N_EXPERTS, D_FF, d),
        full(N_LAYERS, N_EXPERTS, 1, d),
        full(d, HIDDEN),
        full(1, HIDDEN),
        full(HIDDEN, N_CLASSES),
        full(1, N_CLASSES),
    ]
    logits, aux = pl.pallas_call(
        _net_body,
        grid=(GRID,),
        in_specs=in_specs,
        out_specs=[
            pl.BlockSpec((BT, N_CLASSES), lambda i: (i, 0)),
            pl.BlockSpec((1, N_LAYERS), lambda i: (0, 0)),
        ],
        out_shape=[
            jax.ShapeDtypeStruct((B, N_CLASSES), jnp.float32),
            jax.ShapeDtypeStruct((1, N_LAYERS), jnp.float32),
        ],
        scratch_shapes=[pltpu.VMEM((2 * N_LAYERS, N_EXPERTS), jnp.float32)],
    )(
        cont, cate,
        p['Wp'][:N_CONT], p['Wp'][N_CONT:], p['bp'].reshape(1, d),
        p['Wv'], p['Wo'],
        p['ln1_g'].reshape(N_LAYERS, 1, d), p['ln1_b'].reshape(N_LAYERS, 1, d),
        p['ln2_g'].reshape(N_LAYERS, 1, d), p['ln2_b'].reshape(N_LAYERS, 1, d),
        p['Wr'], p['W1'], p['b1'].reshape(N_LAYERS, N_EXPERTS, 1, D_FF),
        p['W2'], p['b2'].reshape(N_LAYERS, N_EXPERTS, 1, d),
        p['Wc1'], p['bc1'].reshape(1, HIDDEN),
        p['Wc2'], p['bc2'].reshape(1, N_CLASSES),
    )
    return logits, aux.reshape(N_LAYERS)


def kernel(cont, params, cat):
    table = params['emb'].reshape(N_CAT * VOCAB, EMB)
    idx = (cat.astype(jnp.int32)
           + (jnp.arange(N_CAT, dtype=jnp.int32) * VOCAB)[None, :]).reshape(-1)
    cate = _embed_gather_sc(table, idx).reshape(B, N_CAT * EMB)
    return _net_tc(cont, cate, params)


# speedup vs baseline: 18.5697x; 1.0579x over previous
import functools

import jax
import jax.numpy as jnp
from jax import lax
from jax.experimental import pallas as pl
from jax.experimental.pallas import tpu as pltpu
from jax.experimental.pallas import tpu_sc as plsc

B = 4096
N_CONT = 13
N_CAT = 26
VOCAB = 1000
EMB = 16
D_MODEL = 128
N_EXPERTS = 8
TOP_K = 2
D_FF = 256
N_LAYERS = 2
N_CLASSES = 2
HIDDEN = max(D_FF, D_MODEL)

BT = 1024
GRID = B // BT
NROWS = B * N_CAT


def _embed_gather_sc(table, idx):
    info = plsc.get_sparse_core_info()
    nw = info.num_cores * info.num_subcores
    b_per_w = NROWS // nw
    nc = info.num_cores
    mesh = plsc.VectorSubcoreMesh(core_axis_name="c", subcore_axis_name="s")

    @functools.partial(
        pl.kernel,
        mesh=mesh,
        out_type=jax.ShapeDtypeStruct((NROWS, EMB), jnp.float32),
        scratch_types=[
            pltpu.VMEM((b_per_w,), jnp.int32),
            pltpu.VMEM((b_per_w, EMB), jnp.float32),
            pltpu.SemaphoreType.DMA,
        ],
        compiler_params=pltpu.CompilerParams(use_tc_tiling_on_sc=False),
    )
    def gather(table_hbm, idx_hbm, out_hbm, idx_v, rows_v, sem):
        wid = lax.axis_index("s") * nc + lax.axis_index("c")
        base = wid * b_per_w
        pltpu.sync_copy(idx_hbm.at[pl.ds(base, b_per_w)], idx_v)
        pltpu.async_copy(table_hbm.at[idx_v], rows_v, sem).wait()
        pltpu.sync_copy(rows_v, out_hbm.at[pl.ds(base, b_per_w)])

    return gather(table, idx)


def _dot(a, b):
    return jax.lax.dot_general(
        a, b, (((a.ndim - 1,), (0,)), ((), ())),
        preferred_element_type=jnp.float32)


def _layer_norm(x, g, b):
    mu = jnp.mean(x, axis=-1, keepdims=True)
    xc = x - mu
    var = jnp.mean(xc * xc, axis=-1, keepdims=True)
    return xc * lax.rsqrt(var + 1e-5) * g + b


def _net_body(cont_ref, cate_ref, wpc_ref, wpe_ref, bp_ref, wv_ref, wo_ref,
              ln1g_ref, ln1b_ref, ln2g_ref, ln2b_ref, wr_ref, w1_ref, b1_ref,
              w2_ref, b2_ref, wc1_ref, bc1_ref, wc2_ref, bc2_ref,
              logits_ref, aux_ref, acc_ref):
    step = pl.program_id(0)

    @pl.when(step == 0)
    def _init():
        acc_ref[...] = jnp.zeros_like(acc_ref)

    z = (_dot(cont_ref[...], wpc_ref[...]) + _dot(cate_ref[...], wpe_ref[...])
         + bp_ref[...])
    for l in range(N_LAYERS):
        attn = _dot(_dot(z, wv_ref[l]), wo_ref[l])
        z = _layer_norm(z + attn, ln1g_ref[l], ln1b_ref[l])
        probs = jax.nn.softmax(_dot(z, wr_ref[l]), axis=-1)
        iota = lax.broadcasted_iota(jnp.int32, probs.shape, 1)
        m1 = jnp.max(probs, axis=-1, keepdims=True)
        i1 = jnp.min(jnp.where(probs == m1, iota, N_EXPERTS), axis=-1,
                     keepdims=True)
        oh1 = iota == i1
        p2 = jnp.where(oh1, -1.0, probs)
        m2 = jnp.max(p2, axis=-1, keepdims=True)
        i2 = jnp.min(jnp.where(p2 == m2, iota, N_EXPERTS), axis=-1,
                     keepdims=True)
        oh2 = iota == i2
        denom = m1 + m2 + 1e-9
        combine = (jnp.where(oh1, m1 / denom, 0.0)
                   + jnp.where(oh2, m2 / denom, 0.0))
        acc_ref[2 * l:2 * l + 1, :] += jnp.sum(probs, axis=0, keepdims=True)
        acc_ref[2 * l + 1:2 * l + 2, :] += jnp.sum(
            (combine > 0).astype(jnp.float32), axis=0, keepdims=True)
        moe = jnp.zeros_like(z)
        for e in range(N_EXPERTS):
            h = jax.nn.gelu(_dot(z, w1_ref[l, e]) + b1_ref[l, e])
            y = _dot(h, w2_ref[l, e]) + b2_ref[l, e]
            moe = moe + y * combine[:, e:e + 1]
        z = _layer_norm(z + moe, ln2g_ref[l], ln2b_ref[l])
    hc = jax.nn.gelu(_dot(z, wc1_ref[...]) + bc1_ref[...])
    logits_ref[...] = _dot(hc, wc2_ref[...]) + bc2_ref[...]

    @pl.when(step == GRID - 1)
    def _fin():
        scale = float(N_EXPERTS) / (B * B)
        a0 = jnp.sum(acc_ref[0:1, :] * acc_ref[1:2, :]) * scale
        a1 = jnp.sum(acc_ref[2:3, :] * acc_ref[3:4, :]) * scale
        aux_ref[...] = jnp.concatenate(
            [jnp.full((1, 1), a0, jnp.float32),
             jnp.full((1, 1), a1, jnp.float32)], axis=1)


def _net_tc(cont, cate, p):
    d = D_MODEL
    full = lambda *shape: pl.BlockSpec(shape, lambda i: (0,) * len(shape))
    in_specs = [
        pl.BlockSpec((BT, N_CONT), lambda i: (i, 0)),
        pl.BlockSpec((BT, N_CAT * EMB), lambda i: (i, 0)),
        full(N_CONT, d),
        full(N_CAT * EMB, d),
        full(1, d),
        full(N_LAYERS, d, d),
        full(N_LAYERS, d, d),
        full(N_LAYERS, 1, d),
        full(N_LAYERS, 1, d),
        full(N_LAYERS, 1, d),
        full(N_LAYERS, 1, d),
        full(N_LAYERS, d, N_EXPERTS),
        full(N_LAYERS, N_EXPERTS, d, D_FF),
        full(N_LAYERS, N_EXPERTS, 1, D_FF),
        full(N_LAYERS, N_EXPERTS, D_FF, d),
        full(N_LAYERS, N_EXPERTS, 1, d),
        full(d, HIDDEN),
        full(1, HIDDEN),
        full(HIDDEN, N_CLASSES),
        full(1, N_CLASSES),
    ]
    logits, aux = pl.pallas_call(
        _net_body,
        grid=(GRID,),
        in_specs=in_specs,
        out_specs=[
            pl.BlockSpec((BT, N_CLASSES), lambda i: (i, 0)),
            pl.BlockSpec((1, N_LAYERS), lambda i: (0, 0)),
        ],
        out_shape=[
            jax.ShapeDtypeStruct((B, N_CLASSES), jnp.float32),
            jax.ShapeDtypeStruct((1, N_LAYERS), jnp.float32),
        ],
        scratch_shapes=[pltpu.VMEM((2 * N_LAYERS, N_EXPERTS), jnp.float32)],
    )(
        cont, cate,
        p['Wp'][:N_CONT], p['Wp'][N_CONT:], p['bp'].reshape(1, d),
        p['Wv'], p['Wo'],
        p['ln1_g'].reshape(N_LAYERS, 1, d), p['ln1_b'].reshape(N_LAYERS, 1, d),
        p['ln2_g'].reshape(N_LAYERS, 1, d), p['ln2_b'].reshape(N_LAYERS, 1, d),
        p['Wr'], p['W1'], p['b1'].reshape(N_LAYERS, N_EXPERTS, 1, D_FF),
        p['W2'], p['b2'].reshape(N_LAYERS, N_EXPERTS, 1, d),
        p['Wc1'], p['bc1'].reshape(1, HIDDEN),
        p['Wc2'], p['bc2'].reshape(1, N_CLASSES),
    )
    return logits, aux.reshape(N_LAYERS)


def kernel(cont, params, cat):
    table = params['emb'].reshape(N_CAT * VOCAB, EMB)
    idx = (cat.astype(jnp.int32)
           + (jnp.arange(N_CAT, dtype=jnp.int32) * VOCAB)[None, :]).reshape(-1)
    cate = _embed_gather_sc(table, idx).reshape(B, N_CAT * EMB)
    return _net_tc(cont, cate, params)


# speedup vs baseline: 19.7543x; 1.0638x over previous
import functools

import jax
import jax.numpy as jnp
from jax import lax
from jax.experimental import pallas as pl
from jax.experimental.pallas import tpu as pltpu
from jax.experimental.pallas import tpu_sc as plsc

B = 4096
N_CONT = 13
N_CAT = 26
VOCAB = 1000
EMB = 16
D_MODEL = 128
N_EXPERTS = 8
TOP_K = 2
D_FF = 256
N_LAYERS = 2
N_CLASSES = 2
HIDDEN = max(D_FF, D_MODEL)

BT = 2048
GRID = B // BT
NROWS = B * N_CAT


def _embed_gather_sc(table, idx):
    info = plsc.get_sparse_core_info()
    nw = info.num_cores * info.num_subcores
    b_per_w = NROWS // nw
    nc = info.num_cores
    mesh = plsc.VectorSubcoreMesh(core_axis_name="c", subcore_axis_name="s")

    @functools.partial(
        pl.kernel,
        mesh=mesh,
        out_type=jax.ShapeDtypeStruct((NROWS, EMB), jnp.float32),
        scratch_types=[
            pltpu.VMEM((b_per_w,), jnp.int32),
            pltpu.VMEM((b_per_w, EMB), jnp.float32),
            pltpu.SemaphoreType.DMA,
        ],
        compiler_params=pltpu.CompilerParams(use_tc_tiling_on_sc=False),
    )
    def gather(table_hbm, idx_hbm, out_hbm, idx_v, rows_v, sem):
        wid = lax.axis_index("s") * nc + lax.axis_index("c")
        base = wid * b_per_w
        pltpu.sync_copy(idx_hbm.at[pl.ds(base, b_per_w)], idx_v)
        pltpu.async_copy(table_hbm.at[idx_v], rows_v, sem).wait()
        pltpu.sync_copy(rows_v, out_hbm.at[pl.ds(base, b_per_w)])

    return gather(table, idx)


def _dot(a, b):
    return jax.lax.dot_general(
        a, b, (((a.ndim - 1,), (0,)), ((), ())),
        preferred_element_type=jnp.float32)


def _layer_norm(x, g, b):
    mu = jnp.mean(x, axis=-1, keepdims=True)
    xc = x - mu
    var = jnp.mean(xc * xc, axis=-1, keepdims=True)
    return xc * lax.rsqrt(var + 1e-5) * g + b


def _net_body(cont_ref, cate_ref, wpc_ref, wpe_ref, bp_ref, wv_ref, wo_ref,
              ln1g_ref, ln1b_ref, ln2g_ref, ln2b_ref, wr_ref, w1_ref, b1_ref,
              w2_ref, b2_ref, wc1_ref, bc1_ref, wc2_ref, bc2_ref,
              logits_ref, aux_ref, acc_ref):
    step = pl.program_id(0)

    @pl.when(step == 0)
    def _init():
        acc_ref[...] = jnp.zeros_like(acc_ref)

    z = (_dot(cont_ref[...], wpc_ref[...]) + _dot(cate_ref[...], wpe_ref[...])
         + bp_ref[...])
    for l in range(N_LAYERS):
        attn = _dot(_dot(z, wv_ref[l]), wo_ref[l])
        z = _layer_norm(z + attn, ln1g_ref[l], ln1b_ref[l])
        probs = jax.nn.softmax(_dot(z, wr_ref[l]), axis=-1)
        iota = lax.broadcasted_iota(jnp.int32, probs.shape, 1)
        m1 = jnp.max(probs, axis=-1, keepdims=True)
        i1 = jnp.min(jnp.where(probs == m1, iota, N_EXPERTS), axis=-1,
                     keepdims=True)
        oh1 = iota == i1
        p2 = jnp.where(oh1, -1.0, probs)
        m2 = jnp.max(p2, axis=-1, keepdims=True)
        i2 = jnp.min(jnp.where(p2 == m2, iota, N_EXPERTS), axis=-1,
                     keepdims=True)
        oh2 = iota == i2
        denom = m1 + m2 + 1e-9
        combine = (jnp.where(oh1, m1 / denom, 0.0)
                   + jnp.where(oh2, m2 / denom, 0.0))
        acc_ref[2 * l:2 * l + 1, :] += jnp.sum(probs, axis=0, keepdims=True)
        acc_ref[2 * l + 1:2 * l + 2, :] += jnp.sum(
            (combine > 0).astype(jnp.float32), axis=0, keepdims=True)
        moe = jnp.zeros_like(z)
        for e in range(N_EXPERTS):
            h = jax.nn.gelu(_dot(z, w1_ref[l, e]) + b1_ref[l, e])
            y = _dot(h, w2_ref[l, e]) + b2_ref[l, e]
            moe = moe + y * combine[:, e:e + 1]
        z = _layer_norm(z + moe, ln2g_ref[l], ln2b_ref[l])
    hc = jax.nn.gelu(_dot(z, wc1_ref[...]) + bc1_ref[...])
    logits_ref[...] = _dot(hc, wc2_ref[...]) + bc2_ref[...]

    @pl.when(step == GRID - 1)
    def _fin():
        scale = float(N_EXPERTS) / (B * B)
        a0 = jnp.sum(acc_ref[0:1, :] * acc_ref[1:2, :]) * scale
        a1 = jnp.sum(acc_ref[2:3, :] * acc_ref[3:4, :]) * scale
        aux_ref[...] = jnp.concatenate(
            [jnp.full((1, 1), a0, jnp.float32),
             jnp.full((1, 1), a1, jnp.float32)], axis=1)


def _net_tc(cont, cate, p):
    d = D_MODEL
    full = lambda *shape: pl.BlockSpec(shape, lambda i: (0,) * len(shape))
    in_specs = [
        pl.BlockSpec((BT, N_CONT), lambda i: (i, 0)),
        pl.BlockSpec((BT, N_CAT * EMB), lambda i: (i, 0)),
        full(N_CONT, d),
        full(N_CAT * EMB, d),
        full(1, d),
        full(N_LAYERS, d, d),
        full(N_LAYERS, d, d),
        full(N_LAYERS, 1, d),
        full(N_LAYERS, 1, d),
        full(N_LAYERS, 1, d),
        full(N_LAYERS, 1, d),
        full(N_LAYERS, d, N_EXPERTS),
        full(N_LAYERS, N_EXPERTS, d, D_FF),
        full(N_LAYERS, N_EXPERTS, 1, D_FF),
        full(N_LAYERS, N_EXPERTS, D_FF, d),
        full(N_LAYERS, N_EXPERTS, 1, d),
        full(d, HIDDEN),
        full(1, HIDDEN),
        full(HIDDEN, N_CLASSES),
        full(1, N_CLASSES),
    ]
    logits, aux = pl.pallas_call(
        _net_body,
        grid=(GRID,),
        in_specs=in_specs,
        out_specs=[
            pl.BlockSpec((BT, N_CLASSES), lambda i: (i, 0)),
            pl.BlockSpec((1, N_LAYERS), lambda i: (0, 0)),
        ],
        out_shape=[
            jax.ShapeDtypeStruct((B, N_CLASSES), jnp.float32),
            jax.ShapeDtypeStruct((1, N_LAYERS), jnp.float32),
        ],
        scratch_shapes=[pltpu.VMEM((2 * N_LAYERS, N_EXPERTS), jnp.float32)],
    )(
        cont, cate,
        p['Wp'][:N_CONT], p['Wp'][N_CONT:], p['bp'].reshape(1, d),
        p['Wv'], p['Wo'],
        p['ln1_g'].reshape(N_LAYERS, 1, d), p['ln1_b'].reshape(N_LAYERS, 1, d),
        p['ln2_g'].reshape(N_LAYERS, 1, d), p['ln2_b'].reshape(N_LAYERS, 1, d),
        p['Wr'], p['W1'], p['b1'].reshape(N_LAYERS, N_EXPERTS, 1, D_FF),
        p['W2'], p['b2'].reshape(N_LAYERS, N_EXPERTS, 1, d),
        p['Wc1'], p['bc1'].reshape(1, HIDDEN),
        p['Wc2'], p['bc2'].reshape(1, N_CLASSES),
    )
    return logits, aux.reshape(N_LAYERS)


def kernel(cont, params, cat):
    table = params['emb'].reshape(N_CAT * VOCAB, EMB)
    idx = (cat.astype(jnp.int32)
           + (jnp.arange(N_CAT, dtype=jnp.int32) * VOCAB)[None, :]).reshape(-1)
    cate = _embed_gather_sc(table, idx).reshape(B, N_CAT * EMB)
    return _net_tc(cont, cate, params)
